# baseline (speedup 1.0000x reference)
import functools

import jax
import jax.numpy as jnp
from jax import lax
from jax.experimental import pallas as pl
from jax.experimental.pallas import tpu as pltpu
from jax.experimental.pallas import tpu_sc as plsc

_NC = 2
_NS = 16
_NW = _NC * _NS
_LANES = 16


def _mesh():
    return plsc.VectorSubcoreMesh(core_axis_name="c", subcore_axis_name="s")


def _wid():
    return lax.axis_index("s") * _NC + lax.axis_index("c")


@functools.lru_cache(maxsize=None)
def _make_table_transpose(v, d):
    assert d == 32
    n_full = v // 128
    tail = v - n_full * 128
    assert (tail * d) % 128 == 0
    per_w = n_full // _NW
    extra = n_full - per_w * _NW
    in_types = [jax.ShapeDtypeStruct((d, v), jnp.float32)]
    if tail:
        in_types.append(
            jax.ShapeDtypeStruct((tail * d // 128, 128), jnp.float32))

    @functools.partial(
        pl.kernel,
        mesh=_mesh(),
        out_type=jax.ShapeDtypeStruct((v * d // 128, 128), jnp.float32),
        scratch_types=[
            pltpu.VMEM((d, 128), jnp.float32),
            pltpu.VMEM((d, 128), jnp.float32),
            pltpu.VMEM((d, 128), jnp.float32),
            pltpu.VMEM((d, 128), jnp.float32),
            pltpu.SemaphoreType.DMA,
            pltpu.SemaphoreType.DMA,
            pltpu.SemaphoreType.DMA,
            pltpu.SemaphoreType.DMA,
        ],
        compiler_params=pltpu.CompilerParams(use_tc_tiling_on_sc=True,
                                             needs_layout_passes=False),
    )
    def tr_kernel(tabt_hbm, *rest):
        if tail:
            tail_hbm, rt_hbm, vin0, vin1, vout0, vout1, rs0, rs1, ws0, ws1 \
                = rest
        else:
            rt_hbm, vin0, vin1, vout0, vout1, rs0, rs1, ws0, ws1 = rest
            tail_hbm = None
        vin = (vin0, vin1)
        vout = (vout0, vout1)
        wid = _wid()
        nblk = per_w + jnp.where(wid < extra, 1, 0)
        start = wid * per_w + jnp.minimum(wid, extra)
        rsems = (rs0, rs1)
        wsems = (ws0, ws1)
        iota = lax.iota(jnp.int32, _LANES)

        def read(bi, slot):
            pltpu.async_copy(
                tabt_hbm.at[:, pl.ds((start + bi) * 128, 128)],
                vin[slot], rsems[slot])

        def transpose(slot):
            def krow(k, carry):
                for q in range(8):
                    rows = (16 * (q % 2)) + iota
                    cols = jnp.full((_LANES,), 4 * k + q // 2, jnp.int32)
                    vals = plsc.load_gather(vin[slot], [rows, cols])
                    vout[slot][k, pl.ds(16 * q, 16)] = vals
                return carry
            lax.fori_loop(0, d, krow, 0)

        def write(bi, slot):
            pltpu.async_copy(
                vout[slot],
                rt_hbm.at[pl.ds((start + bi) * d, d)], wsems[slot])

        read(0, 0)

        def body(bi, carry):
            slot = lax.rem(bi, 2)

            def do(s, first):
                @pl.when((slot == s) & (bi < nblk))
                def _():
                    pltpu.make_async_copy(
                        tabt_hbm.at[:, pl.ds(0, 128)], vin[s], rsems[s]
                    ).wait()

                    @pl.when(bi + 1 < nblk)
                    def _():
                        read(bi + 1, 1 - s)

                    @pl.when(~first)
                    def _():
                        pltpu.make_async_copy(
                            vout[s], rt_hbm.at[pl.ds(0, d)], wsems[s]
                        ).wait()
                    transpose(s)
                    write(bi, s)

            do(0, bi == 0)
            do(1, bi == 1)
            return carry

        lax.fori_loop(0, per_w + 1, body, 0)

        @pl.when(nblk >= 1)
        def _():
            pltpu.make_async_copy(
                vout[0], rt_hbm.at[pl.ds(0, d)], wsems[0]).wait()

        @pl.when(nblk >= 2)
        def _():
            pltpu.make_async_copy(
                vout[1], rt_hbm.at[pl.ds(0, d)], wsems[1]).wait()

        if tail:
            nk = tail * d // 128

            @pl.when(wid == _NW - 1)
            def _():
                pltpu.sync_copy(tail_hbm, vin[0].at[pl.ds(0, nk)])
                pltpu.sync_copy(vin[0].at[pl.ds(0, nk)],
                                rt_hbm.at[pl.ds(n_full * d, nk)])

    return tr_kernel


@functools.lru_cache(maxsize=None)
def _make_gather(b, l, v, d):
    assert d == 32 and b % 128 == 0
    nbblk = b // 128
    units = l * nbblk
    assert units % _NW == 0
    per_w = units // _NW

    @functools.partial(
        pl.kernel,
        mesh=_mesh(),
        out_type=jax.ShapeDtypeStruct((l, d // 8, nbblk, 8, 128),
                                      jnp.float32),
        scratch_types=[
            pltpu.VMEM((2, 128), jnp.int32),
            pltpu.VMEM((2, 128, d), jnp.float32),
            pltpu.VMEM((2, d, 128), jnp.float32),
            pltpu.SemaphoreType.DMA,
            pltpu.SemaphoreType.DMA,
            pltpu.SemaphoreType.DMA,
            pltpu.SemaphoreType.DMA,
        ],
        compiler_params=pltpu.CompilerParams(use_tc_tiling_on_sc=False,
                                             needs_layout_passes=False),
    )
    def g_kernel(tokt_hbm, rt_hbm, out_hbm, idx_v, rows_v, vout,
                 gs0, gs1, ws0, ws1):
        wid = _wid()
        u0 = wid * per_w
        gsems = (gs0, gs1)
        wsems = (ws0, ws1)
        iota = lax.iota(jnp.int32, _LANES)

        def fetch(ui, slot):
            uu = u0 + ui
            li = uu // nbblk
            tc = lax.rem(uu, nbblk)
            pltpu.sync_copy(tokt_hbm.at[li, pl.ds(tc * 128, 128)],
                            idx_v.at[slot])
            pltpu.async_copy(rt_hbm.at[idx_v.at[slot]], rows_v.at[slot],
                             gsems[slot])

        def transpose(slot):
            def crow(c, carry):
                for q in range(8):
                    rows = 16 * q + iota
                    cols = jnp.full((_LANES,), c, jnp.int32)
                    vals = plsc.load_gather(rows_v.at[slot], [rows, cols])
                    vout[slot, c, pl.ds(16 * q, 16)] = vals
                return carry
            lax.fori_loop(0, d, crow, 0)

        def write(ui, slot):
            uu = u0 + ui
            li = uu // nbblk
            tc = lax.rem(uu, nbblk)
            for t in range(d // 8):
                pltpu.async_copy(vout.at[slot, pl.ds(8 * t, 8)],
                                 out_hbm.at[li, t, tc], wsems[slot])

        fetch(0, 0)

        def body(ui, carry):
            slot = lax.rem(ui, 2)

            def do(s):
                @pl.when(slot == s)
                def _():
                    @pl.when(ui + 1 < per_w)
                    def _():
                        fetch(ui + 1, 1 - s)
                    pltpu.make_async_copy(
                        rt_hbm.at[idx_v.at[s]], rows_v.at[s], gsems[s]
                    ).wait()

                    @pl.when(ui >= 2)
                    def _():
                        for t in range(d // 8):
                            pltpu.make_async_copy(
                                vout.at[s, pl.ds(8 * t, 8)],
                                out_hbm.at[0, t, 0], wsems[s]).wait()
                    transpose(s)
                    write(ui, s)

            do(0)
            do(1)
            return carry

        lax.fori_loop(0, per_w, body, 0)

        for s in range(2):
            @pl.when(per_w > s)
            def _():
                for t in range(d // 8):
                    pltpu.make_async_copy(
                        vout.at[s, pl.ds(8 * t, 8)],
                        out_hbm.at[0, t, 0], wsems[s]).wait()

    return g_kernel


def kernel(token_id, table):
    b, l = token_id.shape
    v, d = table.shape
    if token_id.dtype != jnp.int32:
        token_id = token_id.astype(jnp.int32)
    n_full = v // 128
    tail = v - n_full * 128
    args = [table.T]
    if tail:
        args.append(table[n_full * 128:].reshape(tail * d // 128, 128))
    rt = _make_table_transpose(v, d)(*args)
    rt_lin = rt.reshape(v, d)
    out5 = _make_gather(b, l, v, d)(token_id.T, rt_lin)
    return jnp.transpose(out5, (2, 4, 0, 1, 3)).reshape(b, l, d)


# speedup vs baseline: 1.0623x; 1.0623x over previous
import functools

import jax
import jax.numpy as jnp
from jax import lax
from jax.experimental import pallas as pl
from jax.experimental.pallas import tpu as pltpu
from jax.experimental.pallas import tpu_sc as plsc

_NC = 2
_NS = 16
_NW = _NC * _NS
_LANES = 16


def _mesh():
    return plsc.VectorSubcoreMesh(core_axis_name="c", subcore_axis_name="s")


def _wid():
    return lax.axis_index("s") * _NC + lax.axis_index("c")


@functools.lru_cache(maxsize=None)
def _make_table_transpose(v, d):
    assert d == 32
    n_full = v // 128
    tail = v - n_full * 128
    assert (tail * d) % 128 == 0
    per_w = n_full // _NW
    extra = n_full - per_w * _NW
    in_types = [jax.ShapeDtypeStruct((d, v), jnp.float32)]
    if tail:
        in_types.append(
            jax.ShapeDtypeStruct((tail * d // 128, 128), jnp.float32))

    @functools.partial(
        pl.kernel,
        mesh=_mesh(),
        out_type=jax.ShapeDtypeStruct((v * d // 128, 128), jnp.float32),
        scratch_types=[
            pltpu.VMEM((d, 128), jnp.float32),
            pltpu.VMEM((d, 128), jnp.float32),
            pltpu.VMEM((d, 128), jnp.float32),
            pltpu.VMEM((d, 128), jnp.float32),
            pltpu.SemaphoreType.DMA,
            pltpu.SemaphoreType.DMA,
            pltpu.SemaphoreType.DMA,
            pltpu.SemaphoreType.DMA,
        ],
        compiler_params=pltpu.CompilerParams(use_tc_tiling_on_sc=True,
                                             needs_layout_passes=False),
    )
    def tr_kernel(tabt_hbm, *rest):
        if tail:
            tail_hbm, rt_hbm, vin0, vin1, vout0, vout1, rs0, rs1, ws0, ws1 \
                = rest
        else:
            rt_hbm, vin0, vin1, vout0, vout1, rs0, rs1, ws0, ws1 = rest
            tail_hbm = None
        vin = (vin0, vin1)
        vout = (vout0, vout1)
        wid = _wid()
        nblk = per_w + jnp.where(wid < extra, 1, 0)
        start = wid * per_w + jnp.minimum(wid, extra)
        rsems = (rs0, rs1)
        wsems = (ws0, ws1)
        iota = lax.iota(jnp.int32, _LANES)

        def read(bi, slot):
            pltpu.async_copy(
                tabt_hbm.at[:, pl.ds((start + bi) * 128, 128)],
                vin[slot], rsems[slot])

        def transpose(slot):
            def krow(k, carry):
                for q in range(8):
                    rows = (16 * (q % 2)) + iota
                    cols = jnp.full((_LANES,), 4 * k + q // 2, jnp.int32)
                    vals = plsc.load_gather(vin[slot], [rows, cols])
                    vout[slot][k, pl.ds(16 * q, 16)] = vals
                return carry
            lax.fori_loop(0, d, krow, 0)

        def write(bi, slot):
            pltpu.async_copy(
                vout[slot],
                rt_hbm.at[pl.ds((start + bi) * d, d)], wsems[slot])

        read(0, 0)

        def body(bi, carry):
            slot = lax.rem(bi, 2)

            def do(s, first):
                @pl.when((slot == s) & (bi < nblk))
                def _():
                    pltpu.make_async_copy(
                        tabt_hbm.at[:, pl.ds(0, 128)], vin[s], rsems[s]
                    ).wait()

                    @pl.when(bi + 1 < nblk)
                    def _():
                        read(bi + 1, 1 - s)

                    @pl.when(~first)
                    def _():
                        pltpu.make_async_copy(
                            vout[s], rt_hbm.at[pl.ds(0, d)], wsems[s]
                        ).wait()
                    transpose(s)
                    write(bi, s)

            do(0, bi == 0)
            do(1, bi == 1)
            return carry

        lax.fori_loop(0, per_w + 1, body, 0)

        @pl.when(nblk >= 1)
        def _():
            pltpu.make_async_copy(
                vout[0], rt_hbm.at[pl.ds(0, d)], wsems[0]).wait()

        @pl.when(nblk >= 2)
        def _():
            pltpu.make_async_copy(
                vout[1], rt_hbm.at[pl.ds(0, d)], wsems[1]).wait()

        if tail:
            nk = tail * d // 128

            @pl.when(wid == _NW - 1)
            def _():
                pltpu.sync_copy(tail_hbm, vin[0].at[pl.ds(0, nk)])
                pltpu.sync_copy(vin[0].at[pl.ds(0, nk)],
                                rt_hbm.at[pl.ds(n_full * d, nk)])

    return tr_kernel


_NBUF = 6


@functools.lru_cache(maxsize=None)
def _make_gather(b, l, v, d):
    assert d == 32 and b % 128 == 0
    nbblk = b // 128
    units = l * nbblk
    assert units % _NW == 0
    per_w = units // _NW
    assert per_w > _NBUF

    @functools.partial(
        pl.kernel,
        mesh=_mesh(),
        out_type=jax.ShapeDtypeStruct((l, d // 8, nbblk, 8, 128),
                                      jnp.float32),
        scratch_types=[
            pltpu.VMEM((per_w * 128,), jnp.int32),
            pltpu.VMEM((_NBUF * 128, d), jnp.float32),
            pltpu.VMEM((2 * d, 128), jnp.float32),
            pltpu.SemaphoreType.DMA((_NBUF,)),
            pltpu.SemaphoreType.DMA,
            pltpu.SemaphoreType.DMA,
        ],
        compiler_params=pltpu.CompilerParams(use_tc_tiling_on_sc=False,
                                             needs_layout_passes=False),
    )
    def g_kernel(tokf_hbm, rt_hbm, out_hbm, ids_v, rows_v, vout,
                 gsem, ws0, ws1):
        wid = _wid()
        u0 = wid * per_w
        wsems = (ws0, ws1)
        iota = lax.iota(jnp.int32, _LANES)

        pltpu.sync_copy(tokf_hbm.at[pl.ds(u0 * 128, per_w * 128)], ids_v)

        def issue(ui):
            s = lax.rem(ui, _NBUF)
            pltpu.async_copy(rt_hbm.at[ids_v.at[pl.ds(ui * 128, 128)]],
                             rows_v.at[pl.ds(s * 128, 128)], gsem.at[s])

        for i in range(_NBUF - 1):
            issue(i)

        def body(ui, carry):
            s6 = lax.rem(ui, _NBUF)
            s2 = lax.rem(ui, 2)
            uu = u0 + ui
            li = uu // nbblk
            tc = lax.rem(uu, nbblk)

            pltpu.make_async_copy(
                rt_hbm.at[ids_v.at[pl.ds(0, 128)]],
                rows_v.at[pl.ds(0, 128)], gsem.at[s6]).wait()

            @pl.when(ui + _NBUF - 1 < per_w)
            def _():
                issue(ui + _NBUF - 1)

            def wait_writes(sem):
                for t in range(d // 8):
                    pltpu.make_async_copy(vout.at[pl.ds(0, 8)],
                                          out_hbm.at[0, t, 0], sem).wait()

            @pl.when((ui >= 2) & (s2 == 0))
            def _():
                wait_writes(ws0)

            @pl.when((ui >= 2) & (s2 == 1))
            def _():
                wait_writes(ws1)

            def crow(c, carry2):
                for q in range(8):
                    rows = s6 * 128 + 16 * q + iota
                    cols = jnp.full((_LANES,), c, jnp.int32)
                    vals = plsc.load_gather(rows_v, [rows, cols])
                    vout[s2 * d + c, pl.ds(16 * q, 16)] = vals
                return carry2
            lax.fori_loop(0, d, crow, 0)

            def write(sem):
                for t in range(d // 8):
                    pltpu.async_copy(vout.at[pl.ds(s2 * d + 8 * t, 8)],
                                     out_hbm.at[li, t, tc], sem)

            @pl.when(s2 == 0)
            def _():
                write(ws0)

            @pl.when(s2 == 1)
            def _():
                write(ws1)
            return carry

        lax.fori_loop(0, per_w, body, 0)

        for s in range(2):
            for t in range(d // 8):
                pltpu.make_async_copy(vout.at[pl.ds(0, 8)],
                                      out_hbm.at[0, t, 0], wsems[s]).wait()

    return g_kernel


def kernel(token_id, table):
    b, l = token_id.shape
    v, d = table.shape
    if token_id.dtype != jnp.int32:
        token_id = token_id.astype(jnp.int32)
    n_full = v // 128
    tail = v - n_full * 128
    args = [table.T]
    if tail:
        args.append(table[n_full * 128:].reshape(tail * d // 128, 128))
    rt = _make_table_transpose(v, d)(*args)
    rt_lin = rt.reshape(v, d)
    out5 = _make_gather(b, l, v, d)(token_id.T.reshape(b * l), rt_lin)
    return jnp.transpose(out5, (2, 4, 0, 1, 3)).reshape(b, l, d)


# speedup vs baseline: 1.3150x; 1.2378x over previous
import functools

import jax
import jax.numpy as jnp
from jax import lax
from jax.experimental import pallas as pl
from jax.experimental.pallas import tpu as pltpu
from jax.experimental.pallas import tpu_sc as plsc

_NC = 2
_NS = 16
_NW = _NC * _NS
_LANES = 16


def _mesh():
    return plsc.VectorSubcoreMesh(core_axis_name="c", subcore_axis_name="s")


def _wid():
    return lax.axis_index("s") * _NC + lax.axis_index("c")


@functools.lru_cache(maxsize=None)
def _make_table_transpose(v, d):
    assert d == 32
    n_full = v // 128
    tail = v - n_full * 128
    assert (tail * d) % 128 == 0
    per_w = n_full // _NW
    extra = n_full - per_w * _NW
    in_types = [jax.ShapeDtypeStruct((d, v), jnp.float32)]
    if tail:
        in_types.append(
            jax.ShapeDtypeStruct((tail * d // 128, 128), jnp.float32))

    @functools.partial(
        pl.kernel,
        mesh=_mesh(),
        out_type=jax.ShapeDtypeStruct((v * d // 128, 128), jnp.float32),
        scratch_types=[
            pltpu.VMEM((d, 128), jnp.float32),
            pltpu.VMEM((d, 128), jnp.float32),
            pltpu.VMEM((d, 128), jnp.float32),
            pltpu.VMEM((d, 128), jnp.float32),
            pltpu.SemaphoreType.DMA,
            pltpu.SemaphoreType.DMA,
            pltpu.SemaphoreType.DMA,
            pltpu.SemaphoreType.DMA,
        ],
        compiler_params=pltpu.CompilerParams(use_tc_tiling_on_sc=True,
                                             needs_layout_passes=False),
    )
    def tr_kernel(tabt_hbm, *rest):
        if tail:
            tail_hbm, rt_hbm, vin0, vin1, vout0, vout1, rs0, rs1, ws0, ws1 \
                = rest
        else:
            rt_hbm, vin0, vin1, vout0, vout1, rs0, rs1, ws0, ws1 = rest
            tail_hbm = None
        vin = (vin0, vin1)
        vout = (vout0, vout1)
        wid = _wid()
        nblk = per_w + jnp.where(wid < extra, 1, 0)
        start = wid * per_w + jnp.minimum(wid, extra)
        rsems = (rs0, rs1)
        wsems = (ws0, ws1)
        iota = lax.iota(jnp.int32, _LANES)

        def read(bi, slot):
            pltpu.async_copy(
                tabt_hbm.at[:, pl.ds((start + bi) * 128, 128)],
                vin[slot], rsems[slot])

        def transpose(slot):
            def krow(k, carry):
                for q in range(8):
                    rows = (16 * (q % 2)) + iota
                    cols = jnp.full((_LANES,), 4 * k + q // 2, jnp.int32)
                    vals = plsc.load_gather(vin[slot], [rows, cols])
                    vout[slot][k, pl.ds(16 * q, 16)] = vals
                return carry
            lax.fori_loop(0, d, krow, 0)

        def write(bi, slot):
            pltpu.async_copy(
                vout[slot],
                rt_hbm.at[pl.ds((start + bi) * d, d)], wsems[slot])

        read(0, 0)

        def body(bi, carry):
            slot = lax.rem(bi, 2)

            def do(s, first):
                @pl.when((slot == s) & (bi < nblk))
                def _():
                    pltpu.make_async_copy(
                        tabt_hbm.at[:, pl.ds(0, 128)], vin[s], rsems[s]
                    ).wait()

                    @pl.when(bi + 1 < nblk)
                    def _():
                        read(bi + 1, 1 - s)

                    @pl.when(~first)
                    def _():
                        pltpu.make_async_copy(
                            vout[s], rt_hbm.at[pl.ds(0, d)], wsems[s]
                        ).wait()
                    transpose(s)
                    write(bi, s)

            do(0, bi == 0)
            do(1, bi == 1)
            return carry

        lax.fori_loop(0, per_w + 1, body, 0)

        @pl.when(nblk >= 1)
        def _():
            pltpu.make_async_copy(
                vout[0], rt_hbm.at[pl.ds(0, d)], wsems[0]).wait()

        @pl.when(nblk >= 2)
        def _():
            pltpu.make_async_copy(
                vout[1], rt_hbm.at[pl.ds(0, d)], wsems[1]).wait()

        if tail:
            nk = tail * d // 128

            @pl.when(wid == _NW - 1)
            def _():
                pltpu.sync_copy(tail_hbm, vin[0].at[pl.ds(0, nk)])
                pltpu.sync_copy(vin[0].at[pl.ds(0, nk)],
                                rt_hbm.at[pl.ds(n_full * d, nk)])

    return tr_kernel


_NBUF = 6


@functools.lru_cache(maxsize=None)
def _make_gather(b, l, v, d):
    assert d == 32 and b % 128 == 0
    nbblk = b // 128
    units = l * nbblk
    assert units % _NW == 0
    per_w = units // _NW
    assert per_w > _NBUF

    @functools.partial(
        pl.kernel,
        mesh=_mesh(),
        out_type=jax.ShapeDtypeStruct((l, d // 8, nbblk, 8, 128),
                                      jnp.float32),
        scratch_types=[
            pltpu.VMEM((per_w * 128,), jnp.int32),
            pltpu.VMEM((_NBUF * 128, d), jnp.float32),
            pltpu.VMEM((2 * d, 128), jnp.float32),
            pltpu.SemaphoreType.DMA((_NBUF,)),
            pltpu.SemaphoreType.DMA,
            pltpu.SemaphoreType.DMA,
        ],
        compiler_params=pltpu.CompilerParams(use_tc_tiling_on_sc=False,
                                             needs_layout_passes=False),
    )
    def g_kernel(tokf_hbm, rt_hbm, out_hbm, ids_v, rows_v, vout,
                 gsem, ws0, ws1):
        wid = _wid()
        u0 = wid * per_w
        wsems = (ws0, ws1)
        iota = lax.iota(jnp.int32, _LANES)

        pltpu.sync_copy(tokf_hbm.at[pl.ds(u0 * 128, per_w * 128)], ids_v)

        def issue(ui):
            s = lax.rem(ui, _NBUF)
            pltpu.async_copy(rt_hbm.at[ids_v.at[pl.ds(ui * 128, 128)]],
                             rows_v.at[pl.ds(s * 128, 128)], gsem.at[s])

        for i in range(_NBUF - 1):
            issue(i)

        def body(ui, carry):
            s6 = lax.rem(ui, _NBUF)
            s2 = lax.rem(ui, 2)
            uu = u0 + ui
            li = uu // nbblk
            tc = lax.rem(uu, nbblk)

            pltpu.make_async_copy(
                rt_hbm.at[ids_v.at[pl.ds(0, 128)]],
                rows_v.at[pl.ds(0, 128)], gsem.at[s6]).wait()

            @pl.when(ui + _NBUF - 1 < per_w)
            def _():
                issue(ui + _NBUF - 1)

            def wait_writes(sem):
                for t in range(d // 8):
                    pltpu.make_async_copy(vout.at[pl.ds(0, 8)],
                                          out_hbm.at[0, t, 0], sem).wait()

            @pl.when((ui >= 2) & (s2 == 0))
            def _():
                wait_writes(ws0)

            @pl.when((ui >= 2) & (s2 == 1))
            def _():
                wait_writes(ws1)

            def crow(c, carry2):
                for q in range(8):
                    rows = s6 * 128 + 16 * q + iota
                    cols = jnp.full((_LANES,), c, jnp.int32)
                    vals = plsc.load_gather(rows_v, [rows, cols])
                    vout[s2 * d + c, pl.ds(16 * q, 16)] = vals
                return carry2
            lax.fori_loop(0, d, crow, 0)

            def write(sem):
                for t in range(d // 8):
                    pltpu.async_copy(vout.at[pl.ds(s2 * d + 8 * t, 8)],
                                     out_hbm.at[li, t, tc], sem)

            @pl.when(s2 == 0)
            def _():
                write(ws0)

            @pl.when(s2 == 1)
            def _():
                write(ws1)
            return carry

        lax.fori_loop(0, per_w, body, 0)

        for s in range(2):
            for t in range(d // 8):
                pltpu.make_async_copy(vout.at[pl.ds(0, 8)],
                                      out_hbm.at[0, t, 0], wsems[s]).wait()

    return g_kernel


def kernel(token_id, table):
    b, l = token_id.shape
    v, d = table.shape
    if token_id.dtype != jnp.int32:
        token_id = token_id.astype(jnp.int32)
    out5 = _make_gather(b, l, v, d)(token_id.T.reshape(b * l), table)
    return jnp.transpose(out5, (2, 4, 0, 1, 3)).reshape(b, l, d)


# speedup vs baseline: 2.0363x; 1.5485x over previous
import functools

import jax
import jax.numpy as jnp
from jax import lax
from jax.experimental import pallas as pl
from jax.experimental.pallas import tpu as pltpu
from jax.experimental.pallas import tpu_sc as plsc

_NC = 2
_NS = 16
_NW = _NC * _NS
_LANES = 16


def _mesh():
    return plsc.VectorSubcoreMesh(core_axis_name="c", subcore_axis_name="s")


def _wid():
    return lax.axis_index("s") * _NC + lax.axis_index("c")


@functools.lru_cache(maxsize=None)
def _make_tc_transpose(v, d, nblk=12800):
    assert nblk % 128 == 0 and (nblk * d) % 128 == 0
    grid = (v + nblk - 1) // nblk

    def body(x_ref, o_ref):
        y = x_ref[...].T
        for r in range(128 // d):
            o_ref[:, d * r:d * (r + 1)] = lax.slice(
                y, (r, 0), (nblk, d), (128 // d, 1))

    return pl.pallas_call(
        body,
        grid=(grid,),
        in_specs=[pl.BlockSpec((d, nblk), lambda j: (0, j))],
        out_specs=pl.BlockSpec((nblk * d // 128, 128), lambda j: (j, 0)),
        out_shape=jax.ShapeDtypeStruct((v * d // 128, 128), jnp.float32),
    )


@functools.lru_cache(maxsize=None)
def _make_table_transpose(v, d):
    assert d == 32
    n_full = v // 128
    tail = v - n_full * 128
    assert (tail * d) % 128 == 0
    per_w = n_full // _NW
    extra = n_full - per_w * _NW
    in_types = [jax.ShapeDtypeStruct((d, v), jnp.float32)]
    if tail:
        in_types.append(
            jax.ShapeDtypeStruct((tail * d // 128, 128), jnp.float32))

    @functools.partial(
        pl.kernel,
        mesh=_mesh(),
        out_type=jax.ShapeDtypeStruct((v * d // 128, 128), jnp.float32),
        scratch_types=[
            pltpu.VMEM((d, 128), jnp.float32),
            pltpu.VMEM((d, 128), jnp.float32),
            pltpu.VMEM((d, 128), jnp.float32),
            pltpu.VMEM((d, 128), jnp.float32),
            pltpu.SemaphoreType.DMA,
            pltpu.SemaphoreType.DMA,
            pltpu.SemaphoreType.DMA,
            pltpu.SemaphoreType.DMA,
        ],
        compiler_params=pltpu.CompilerParams(use_tc_tiling_on_sc=True,
                                             needs_layout_passes=False),
    )
    def tr_kernel(tabt_hbm, *rest):
        if tail:
            tail_hbm, rt_hbm, vin0, vin1, vout0, vout1, rs0, rs1, ws0, ws1 \
                = rest
        else:
            rt_hbm, vin0, vin1, vout0, vout1, rs0, rs1, ws0, ws1 = rest
            tail_hbm = None
        vin = (vin0, vin1)
        vout = (vout0, vout1)
        wid = _wid()
        nblk = per_w + jnp.where(wid < extra, 1, 0)
        start = wid * per_w + jnp.minimum(wid, extra)
        rsems = (rs0, rs1)
        wsems = (ws0, ws1)
        iota = lax.iota(jnp.int32, _LANES)

        def read(bi, slot):
            pltpu.async_copy(
                tabt_hbm.at[:, pl.ds((start + bi) * 128, 128)],
                vin[slot], rsems[slot])

        def transpose(slot):
            def krow(k, carry):
                for q in range(8):
                    rows = (16 * (q % 2)) + iota
                    cols = jnp.full((_LANES,), 4 * k + q // 2, jnp.int32)
                    vals = plsc.load_gather(vin[slot], [rows, cols])
                    vout[slot][k, pl.ds(16 * q, 16)] = vals
                return carry
            lax.fori_loop(0, d, krow, 0)

        def write(bi, slot):
            pltpu.async_copy(
                vout[slot],
                rt_hbm.at[pl.ds((start + bi) * d, d)], wsems[slot])

        read(0, 0)

        def body(bi, carry):
            slot = lax.rem(bi, 2)

            def do(s, first):
                @pl.when((slot == s) & (bi < nblk))
                def _():
                    pltpu.make_async_copy(
                        tabt_hbm.at[:, pl.ds(0, 128)], vin[s], rsems[s]
                    ).wait()

                    @pl.when(bi + 1 < nblk)
                    def _():
                        read(bi + 1, 1 - s)

                    @pl.when(~first)
                    def _():
                        pltpu.make_async_copy(
                            vout[s], rt_hbm.at[pl.ds(0, d)], wsems[s]
                        ).wait()
                    transpose(s)
                    write(bi, s)

            do(0, bi == 0)
            do(1, bi == 1)
            return carry

        lax.fori_loop(0, per_w + 1, body, 0)

        @pl.when(nblk >= 1)
        def _():
            pltpu.make_async_copy(
                vout[0], rt_hbm.at[pl.ds(0, d)], wsems[0]).wait()

        @pl.when(nblk >= 2)
        def _():
            pltpu.make_async_copy(
                vout[1], rt_hbm.at[pl.ds(0, d)], wsems[1]).wait()

        if tail:
            nk = tail * d // 128

            @pl.when(wid == _NW - 1)
            def _():
                pltpu.sync_copy(tail_hbm, vin[0].at[pl.ds(0, nk)])
                pltpu.sync_copy(vin[0].at[pl.ds(0, nk)],
                                rt_hbm.at[pl.ds(n_full * d, nk)])

    return tr_kernel


_NBUF = 6


@functools.lru_cache(maxsize=None)
def _make_gather(b, l, v, d):
    assert d == 32 and b % 128 == 0
    nbblk = b // 128
    units = l * nbblk
    assert units % _NW == 0
    per_w = units // _NW
    assert per_w > _NBUF

    @functools.partial(
        pl.kernel,
        mesh=_mesh(),
        out_type=jax.ShapeDtypeStruct((l, d // 8, nbblk, 8, 128),
                                      jnp.float32),
        scratch_types=[
            pltpu.VMEM((per_w * 128,), jnp.int32),
            pltpu.VMEM((_NBUF * 128, d), jnp.float32),
            pltpu.VMEM((2 * d, 130), jnp.float32),
            pltpu.SemaphoreType.DMA((_NBUF,)),
            pltpu.SemaphoreType.DMA,
            pltpu.SemaphoreType.DMA,
        ],
        compiler_params=pltpu.CompilerParams(use_tc_tiling_on_sc=False,
                                             needs_layout_passes=False),
    )
    def g_kernel(tokf_hbm, rt_hbm, out_hbm, ids_v, rows_v, vout,
                 gsem, ws0, ws1):
        wid = _wid()
        u0 = wid * per_w
        wsems = (ws0, ws1)
        iota = lax.iota(jnp.int32, _LANES)

        pltpu.sync_copy(tokf_hbm.at[pl.ds(u0 * 128, per_w * 128)], ids_v)

        def issue(ui):
            s = lax.rem(ui, _NBUF)
            pltpu.async_copy(rt_hbm.at[ids_v.at[pl.ds(ui * 128, 128)]],
                             rows_v.at[pl.ds(s * 128, 128)], gsem.at[s])

        for i in range(_NBUF - 1):
            issue(i)

        def body(ui, carry):
            s6 = lax.rem(ui, _NBUF)
            s2 = lax.rem(ui, 2)
            uu = u0 + ui
            li = uu // nbblk
            tc = lax.rem(uu, nbblk)

            pltpu.make_async_copy(
                rt_hbm.at[ids_v.at[pl.ds(0, 128)]],
                rows_v.at[pl.ds(0, 128)], gsem.at[s6]).wait()

            @pl.when(ui + _NBUF - 1 < per_w)
            def _():
                issue(ui + _NBUF - 1)

            def wait_writes(sem):
                for t in range(d // 8):
                    pltpu.make_async_copy(
                        vout.at[pl.ds(0, 8), pl.ds(0, 128)],
                        out_hbm.at[0, t, 0], sem).wait()

            @pl.when((ui >= 2) & (s2 == 0))
            def _():
                wait_writes(ws0)

            @pl.when((ui >= 2) & (s2 == 1))
            def _():
                wait_writes(ws1)

            rowsq = [s6 * 128 + 16 * q + iota for q in range(8)]
            colq = [16 * q + iota for q in range(8)]

            def crow(c, carry2):
                colrot = jnp.bitwise_and(c + iota, d - 1)
                srow = colrot + s2 * d
                for q in range(8):
                    vals = plsc.load_gather(rows_v, [rowsq[q], colrot])
                    plsc.store_scatter(vout, [srow, colq[q]], vals)
                return carry2
            lax.fori_loop(0, d, crow, 0)

            def write(sem):
                for t in range(d // 8):
                    pltpu.async_copy(
                        vout.at[pl.ds(s2 * d + 8 * t, 8), pl.ds(0, 128)],
                        out_hbm.at[li, t, tc], sem)

            @pl.when(s2 == 0)
            def _():
                write(ws0)

            @pl.when(s2 == 1)
            def _():
                write(ws1)
            return carry

        lax.fori_loop(0, per_w, body, 0)

        for s in range(2):
            for t in range(d // 8):
                pltpu.make_async_copy(
                    vout.at[pl.ds(0, 8), pl.ds(0, 128)],
                    out_hbm.at[0, t, 0], wsems[s]).wait()

    return g_kernel


def kernel(token_id, table):
    b, l = token_id.shape
    v, d = table.shape
    if token_id.dtype != jnp.int32:
        token_id = token_id.astype(jnp.int32)
    out5 = _make_gather(b, l, v, d)(token_id.T.reshape(b * l), table)
    return jnp.transpose(out5, (2, 4, 0, 1, 3)).reshape(b, l, d)


# speedup vs baseline: 2.8969x; 1.4226x over previous
import functools

import jax
import jax.numpy as jnp
from jax import lax
from jax.experimental import pallas as pl
from jax.experimental.pallas import tpu as pltpu
from jax.experimental.pallas import tpu_sc as plsc

_NC = 2
_NS = 16
_NW = _NC * _NS
_LANES = 16


def _mesh():
    return plsc.VectorSubcoreMesh(core_axis_name="c", subcore_axis_name="s")


def _wid():
    return lax.axis_index("s") * _NC + lax.axis_index("c")


_RNB = 4


@functools.lru_cache(maxsize=None)
def _make_table_transpose(v, d):
    assert d == 32
    n_full = v // 128
    tail = v - n_full * 128
    assert (tail * d) % 128 == 0
    per_w = n_full // _NW
    extra = n_full - per_w * _NW

    @functools.partial(
        pl.kernel,
        mesh=_mesh(),
        out_type=jax.ShapeDtypeStruct((v * d // 128, 128), jnp.float32),
        scratch_types=[
            pltpu.VMEM((_RNB * d, 128), jnp.float32),
            pltpu.VMEM((2 * d, 130), jnp.float32),
            pltpu.SemaphoreType.DMA((_RNB,)),
            pltpu.SemaphoreType.DMA,
            pltpu.SemaphoreType.DMA,
        ],
        compiler_params=pltpu.CompilerParams(use_tc_tiling_on_sc=True,
                                             needs_layout_passes=False),
    )
    def tr_kernel(tabt_hbm, *rest):
        if tail:
            tail_hbm, rt_hbm, vin, vout, rsem, ws0, ws1 = rest
        else:
            rt_hbm, vin, vout, rsem, ws0, ws1 = rest
            tail_hbm = None
        wid = _wid()
        nblk = per_w + jnp.where(wid < extra, 1, 0)
        start = wid * per_w + jnp.minimum(wid, extra)
        wsems = (ws0, ws1)
        iota = lax.iota(jnp.int32, _LANES)

        def read(bi):
            s = lax.rem(bi, _RNB)
            pltpu.async_copy(
                tabt_hbm.at[:, pl.ds((start + bi) * 128, 128)],
                vin.at[pl.ds(s * d, d)], rsem.at[s])

        for i in range(_RNB - 1):
            @pl.when(i < nblk)
            def _():
                read(i)

        def wait_write(sem):
            pltpu.make_async_copy(
                vout.at[pl.ds(0, d), pl.ds(0, 128)],
                rt_hbm.at[pl.ds(0, d)], sem).wait()

        def body(bi, carry):
            @pl.when(bi < nblk)
            def _():
                s4 = lax.rem(bi, _RNB)
                s2 = lax.rem(bi, 2)
                pltpu.make_async_copy(
                    tabt_hbm.at[:, pl.ds(0, 128)],
                    vin.at[pl.ds(0, d)], rsem.at[s4]).wait()

                @pl.when(bi + _RNB - 1 < nblk)
                def _():
                    read(bi + _RNB - 1)

                @pl.when((bi >= 2) & (s2 == 0))
                def _():
                    wait_write(ws0)

                @pl.when((bi >= 2) & (s2 == 1))
                def _():
                    wait_write(ws1)

                colq = [16 * q + iota for q in range(8)]
                srowq = [s2 * d + 4 * q + iota // 4 for q in range(8)]
                mbase = 32 * jnp.bitwise_and(iota, 3)

                def frow(f, c2):
                    rot = jnp.bitwise_and(f + iota, d - 1)
                    grow = s4 * d + rot
                    scol = mbase + rot
                    for q in range(8):
                        vals = plsc.load_gather(vin, [grow, colq[q]])
                        plsc.store_scatter(vout, [srowq[q], scol], vals)
                    return c2
                lax.fori_loop(0, d, frow, 0)

                @pl.when(s2 == 0)
                def _():
                    pltpu.async_copy(
                        vout.at[pl.ds(0, d), pl.ds(0, 128)],
                        rt_hbm.at[pl.ds((start + bi) * d, d)], ws0)

                @pl.when(s2 == 1)
                def _():
                    pltpu.async_copy(
                        vout.at[pl.ds(d, d), pl.ds(0, 128)],
                        rt_hbm.at[pl.ds((start + bi) * d, d)], ws1)
            return carry

        lax.fori_loop(0, per_w + 1, body, 0)

        wait_write(ws0)
        wait_write(ws1)

        if tail:
            nk = tail * d // 128

            @pl.when(wid == _NW - 1)
            def _():
                pltpu.sync_copy(tail_hbm, vin.at[pl.ds(0, nk)])
                pltpu.sync_copy(vin.at[pl.ds(0, nk)],
                                rt_hbm.at[pl.ds(n_full * d, nk)])

    return tr_kernel


_NBUF = 6


@functools.lru_cache(maxsize=None)
def _make_gather(b, l, v, d):
    assert d == 32 and b % 128 == 0
    nbblk = b // 128
    units = l * nbblk
    assert units % _NW == 0
    per_w = units // _NW
    assert per_w > _NBUF

    @functools.partial(
        pl.kernel,
        mesh=_mesh(),
        out_type=jax.ShapeDtypeStruct((l, d // 8, nbblk, 8, 128),
                                      jnp.float32),
        scratch_types=[
            pltpu.VMEM((per_w * 128,), jnp.int32),
            pltpu.VMEM((_NBUF * 128, d), jnp.float32),
            pltpu.VMEM((2 * d, 130), jnp.float32),
            pltpu.SemaphoreType.DMA((_NBUF,)),
            pltpu.SemaphoreType.DMA,
            pltpu.SemaphoreType.DMA,
        ],
        compiler_params=pltpu.CompilerParams(use_tc_tiling_on_sc=False,
                                             needs_layout_passes=False),
    )
    def g_kernel(tokf_hbm, rt_hbm, out_hbm, ids_v, rows_v, vout,
                 gsem, ws0, ws1):
        wid = _wid()
        u0 = wid * per_w
        wsems = (ws0, ws1)
        iota = lax.iota(jnp.int32, _LANES)

        pltpu.sync_copy(tokf_hbm.at[pl.ds(u0 * 128, per_w * 128)], ids_v)

        def issue(ui):
            s = lax.rem(ui, _NBUF)
            pltpu.async_copy(rt_hbm.at[ids_v.at[pl.ds(ui * 128, 128)]],
                             rows_v.at[pl.ds(s * 128, 128)], gsem.at[s])

        for i in range(_NBUF - 1):
            issue(i)

        def body(ui, carry):
            s6 = lax.rem(ui, _NBUF)
            s2 = lax.rem(ui, 2)
            uu = u0 + ui
            li = uu // nbblk
            tc = lax.rem(uu, nbblk)

            pltpu.make_async_copy(
                rt_hbm.at[ids_v.at[pl.ds(0, 128)]],
                rows_v.at[pl.ds(0, 128)], gsem.at[s6]).wait()

            @pl.when(ui + _NBUF - 1 < per_w)
            def _():
                issue(ui + _NBUF - 1)

            def wait_writes(sem):
                for t in range(d // 8):
                    pltpu.make_async_copy(
                        vout.at[pl.ds(0, 8), pl.ds(0, 128)],
                        out_hbm.at[0, t, 0], sem).wait()

            @pl.when((ui >= 2) & (s2 == 0))
            def _():
                wait_writes(ws0)

            @pl.when((ui >= 2) & (s2 == 1))
            def _():
                wait_writes(ws1)

            rowsq = [s6 * 128 + 16 * q + iota for q in range(8)]
            colq = [16 * q + iota for q in range(8)]

            def crow(c, carry2):
                colrot = jnp.bitwise_and(c + iota, d - 1)
                srow = colrot + s2 * d
                for q in range(8):
                    vals = plsc.load_gather(rows_v, [rowsq[q], colrot])
                    plsc.store_scatter(vout, [srow, colq[q]], vals)
                return carry2
            lax.fori_loop(0, d, crow, 0)

            def write(sem):
                for t in range(d // 8):
                    pltpu.async_copy(
                        vout.at[pl.ds(s2 * d + 8 * t, 8), pl.ds(0, 128)],
                        out_hbm.at[li, t, tc], sem)

            @pl.when(s2 == 0)
            def _():
                write(ws0)

            @pl.when(s2 == 1)
            def _():
                write(ws1)
            return carry

        lax.fori_loop(0, per_w, body, 0)

        for s in range(2):
            for t in range(d // 8):
                pltpu.make_async_copy(
                    vout.at[pl.ds(0, 8), pl.ds(0, 128)],
                    out_hbm.at[0, t, 0], wsems[s]).wait()

    return g_kernel


def kernel(token_id, table):
    b, l = token_id.shape
    v, d = table.shape
    if token_id.dtype != jnp.int32:
        token_id = token_id.astype(jnp.int32)
    n_full = v // 128
    tail = v - n_full * 128
    args = [table.T]
    if tail:
        args.append(table[n_full * 128:].reshape(tail * d // 128, 128))
    rt = _make_table_transpose(v, d)(*args)
    rt_lin = rt.reshape(v, d)
    out5 = _make_gather(b, l, v, d)(token_id.T.reshape(b * l), rt_lin)
    return jnp.transpose(out5, (2, 4, 0, 1, 3)).reshape(b, l, d)


# speedup vs baseline: 2.9750x; 1.0270x over previous
import functools

import jax
import jax.numpy as jnp
from jax import lax
from jax.experimental import pallas as pl
from jax.experimental.pallas import tpu as pltpu
from jax.experimental.pallas import tpu_sc as plsc

_NC = 2
_NS = 16
_NW = _NC * _NS
_LANES = 16


def _mesh():
    return plsc.VectorSubcoreMesh(core_axis_name="c", subcore_axis_name="s")


def _wid():
    return lax.axis_index("s") * _NC + lax.axis_index("c")


_RNB = 4


@functools.lru_cache(maxsize=None)
def _make_table_transpose(v, d):
    assert d == 32
    n_full = v // 128
    tail = v - n_full * 128
    assert (tail * d) % 128 == 0
    per_w = n_full // _NW
    extra = n_full - per_w * _NW

    @functools.partial(
        pl.kernel,
        mesh=_mesh(),
        out_type=jax.ShapeDtypeStruct((v * d // 128, 128), jnp.float32),
        scratch_types=[
            pltpu.VMEM((_RNB * d, 128), jnp.float32),
            pltpu.VMEM((2 * d, 130), jnp.float32),
            pltpu.SemaphoreType.DMA((_RNB,)),
            pltpu.SemaphoreType.DMA,
            pltpu.SemaphoreType.DMA,
        ],
        compiler_params=pltpu.CompilerParams(use_tc_tiling_on_sc=True,
                                             needs_layout_passes=False),
    )
    def tr_kernel(tabt_hbm, *rest):
        if tail:
            tail_hbm, rt_hbm, vin, vout, rsem, ws0, ws1 = rest
        else:
            rt_hbm, vin, vout, rsem, ws0, ws1 = rest
            tail_hbm = None
        wid = _wid()
        nblk = per_w + jnp.where(wid < extra, 1, 0)
        start = wid * per_w + jnp.minimum(wid, extra)
        wsems = (ws0, ws1)
        iota = lax.iota(jnp.int32, _LANES)
        colq = [16 * q + iota for q in range(8)]
        base4q = [4 * q + iota // 4 for q in range(8)]
        mbase = 32 * jnp.bitwise_and(iota, 3)

        def read(bi):
            s = lax.rem(bi, _RNB)
            pltpu.async_copy(
                tabt_hbm.at[:, pl.ds((start + bi) * 128, 128)],
                vin.at[pl.ds(s * d, d)], rsem.at[s])

        for i in range(_RNB - 1):
            @pl.when(i < nblk)
            def _():
                read(i)

        def wait_write(sem):
            pltpu.make_async_copy(
                vout.at[pl.ds(0, d), pl.ds(0, 128)],
                rt_hbm.at[pl.ds(0, d)], sem).wait()

        def body(bi, carry):
            @pl.when(bi < nblk)
            def _():
                s4 = lax.rem(bi, _RNB)
                s2 = lax.rem(bi, 2)
                pltpu.make_async_copy(
                    tabt_hbm.at[:, pl.ds(0, 128)],
                    vin.at[pl.ds(0, d)], rsem.at[s4]).wait()

                @pl.when(bi + _RNB - 1 < nblk)
                def _():
                    read(bi + _RNB - 1)

                @pl.when((bi >= 2) & (s2 == 0))
                def _():
                    wait_write(ws0)

                @pl.when((bi >= 2) & (s2 == 1))
                def _():
                    wait_write(ws1)

                srowq = [s2 * d + bq for bq in base4q]

                def frow(f, c2):
                    rot = jnp.bitwise_and(f + iota, d - 1)
                    grow = s4 * d + rot
                    scol = mbase + rot
                    for q in range(8):
                        vals = plsc.load_gather(vin, [grow, colq[q]])
                        plsc.store_scatter(vout, [srowq[q], scol], vals)
                    return c2
                lax.fori_loop(0, d, frow, 0, unroll=2)

                @pl.when(s2 == 0)
                def _():
                    pltpu.async_copy(
                        vout.at[pl.ds(0, d), pl.ds(0, 128)],
                        rt_hbm.at[pl.ds((start + bi) * d, d)], ws0)

                @pl.when(s2 == 1)
                def _():
                    pltpu.async_copy(
                        vout.at[pl.ds(d, d), pl.ds(0, 128)],
                        rt_hbm.at[pl.ds((start + bi) * d, d)], ws1)
            return carry

        lax.fori_loop(0, per_w + 1, body, 0)

        wait_write(ws0)
        wait_write(ws1)

        if tail:
            nk = tail * d // 128

            @pl.when(wid == _NW - 1)
            def _():
                pltpu.sync_copy(tail_hbm, vin.at[pl.ds(0, nk)])
                pltpu.sync_copy(vin.at[pl.ds(0, nk)],
                                rt_hbm.at[pl.ds(n_full * d, nk)])

    return tr_kernel


_NBUF = 6


@functools.lru_cache(maxsize=None)
def _make_gather(b, l, v, d):
    assert d == 32 and b % 128 == 0
    nbblk = b // 128
    units = l * nbblk
    assert units % _NW == 0
    per_w = units // _NW
    assert per_w > _NBUF

    @functools.partial(
        pl.kernel,
        mesh=_mesh(),
        out_type=jax.ShapeDtypeStruct((l, d // 8, nbblk, 8, 128),
                                      jnp.float32),
        scratch_types=[
            pltpu.VMEM((per_w * 128,), jnp.int32),
            pltpu.VMEM((_NBUF * 128, d), jnp.float32),
            pltpu.VMEM((2 * d, 130), jnp.float32),
            pltpu.SemaphoreType.DMA((_NBUF,)),
            pltpu.SemaphoreType.DMA,
            pltpu.SemaphoreType.DMA,
        ],
        compiler_params=pltpu.CompilerParams(use_tc_tiling_on_sc=False,
                                             needs_layout_passes=False),
    )
    def g_kernel(tokf_hbm, rt_hbm, out_hbm, ids_v, rows_v, vout,
                 gsem, ws0, ws1):
        wid = _wid()
        u0 = wid * per_w
        wsems = (ws0, ws1)
        iota = lax.iota(jnp.int32, _LANES)
        colq = [16 * q + iota for q in range(8)]

        pltpu.sync_copy(tokf_hbm.at[pl.ds(u0 * 128, per_w * 128)], ids_v)

        def issue(ui):
            s = lax.rem(ui, _NBUF)
            pltpu.async_copy(rt_hbm.at[ids_v.at[pl.ds(ui * 128, 128)]],
                             rows_v.at[pl.ds(s * 128, 128)], gsem.at[s])

        for i in range(_NBUF - 1):
            issue(i)

        def body(ui, carry):
            s6 = lax.rem(ui, _NBUF)
            s2 = lax.rem(ui, 2)
            uu = u0 + ui
            li = uu // nbblk
            tc = lax.rem(uu, nbblk)

            pltpu.make_async_copy(
                rt_hbm.at[ids_v.at[pl.ds(0, 128)]],
                rows_v.at[pl.ds(0, 128)], gsem.at[s6]).wait()

            @pl.when(ui + _NBUF - 1 < per_w)
            def _():
                issue(ui + _NBUF - 1)

            def wait_writes(sem):
                for t in range(d // 8):
                    pltpu.make_async_copy(
                        vout.at[pl.ds(0, 8), pl.ds(0, 128)],
                        out_hbm.at[0, t, 0], sem).wait()

            @pl.when((ui >= 2) & (s2 == 0))
            def _():
                wait_writes(ws0)

            @pl.when((ui >= 2) & (s2 == 1))
            def _():
                wait_writes(ws1)

            rowsq = [s6 * 128 + cq for cq in colq]

            def crow(c, carry2):
                colrot = jnp.bitwise_and(c + iota, d - 1)
                srow = colrot + s2 * d
                for q in range(8):
                    vals = plsc.load_gather(rows_v, [rowsq[q], colrot])
                    plsc.store_scatter(vout, [srow, colq[q]], vals)
                return carry2
            lax.fori_loop(0, d, crow, 0, unroll=2)

            def write(sem):
                for t in range(d // 8):
                    pltpu.async_copy(
                        vout.at[pl.ds(s2 * d + 8 * t, 8), pl.ds(0, 128)],
                        out_hbm.at[li, t, tc], sem)

            @pl.when(s2 == 0)
            def _():
                write(ws0)

            @pl.when(s2 == 1)
            def _():
                write(ws1)
            return carry

        lax.fori_loop(0, per_w, body, 0)

        for s in range(2):
            for t in range(d // 8):
                pltpu.make_async_copy(
                    vout.at[pl.ds(0, 8), pl.ds(0, 128)],
                    out_hbm.at[0, t, 0], wsems[s]).wait()

    return g_kernel


def kernel(token_id, table):
    b, l = token_id.shape
    v, d = table.shape
    if token_id.dtype != jnp.int32:
        token_id = token_id.astype(jnp.int32)
    n_full = v // 128
    tail = v - n_full * 128
    args = [table.T]
    if tail:
        args.append(table[n_full * 128:].reshape(tail * d // 128, 128))
    rt = _make_table_transpose(v, d)(*args)
    rt_lin = rt.reshape(v, d)
    out5 = _make_gather(b, l, v, d)(token_id.T.reshape(b * l), rt_lin)
    return jnp.transpose(out5, (2, 4, 0, 1, 3)).reshape(b, l, d)


# speedup vs baseline: 3.0382x; 1.0213x over previous
import functools

import jax
import jax.numpy as jnp
from jax import lax
from jax.experimental import pallas as pl
from jax.experimental.pallas import tpu as pltpu
from jax.experimental.pallas import tpu_sc as plsc

_NC = 2
_NS = 16
_NW = _NC * _NS
_LANES = 16


def _mesh():
    return plsc.VectorSubcoreMesh(core_axis_name="c", subcore_axis_name="s")


def _wid():
    return lax.axis_index("s") * _NC + lax.axis_index("c")


_RNB = 4


@functools.lru_cache(maxsize=None)
def _make_table_transpose(v, d):
    assert d == 32
    n_full = v // 128
    tail = v - n_full * 128
    assert (tail * d) % 128 == 0
    per_w = n_full // _NW
    extra = n_full - per_w * _NW

    @functools.partial(
        pl.kernel,
        mesh=_mesh(),
        out_type=jax.ShapeDtypeStruct((v * d // 128, 128), jnp.float32),
        scratch_types=[
            pltpu.VMEM((_RNB * d, 128), jnp.float32),
            pltpu.VMEM((2 * d, 130), jnp.float32),
            pltpu.SemaphoreType.DMA((_RNB,)),
            pltpu.SemaphoreType.DMA,
            pltpu.SemaphoreType.DMA,
        ],
        compiler_params=pltpu.CompilerParams(use_tc_tiling_on_sc=True,
                                             needs_layout_passes=False),
    )
    def tr_kernel(tabt_hbm, *rest):
        if tail:
            tail_hbm, rt_hbm, vin, vout, rsem, ws0, ws1 = rest
        else:
            rt_hbm, vin, vout, rsem, ws0, ws1 = rest
            tail_hbm = None
        wid = _wid()
        nblk = per_w + jnp.where(wid < extra, 1, 0)
        start = wid * per_w + jnp.minimum(wid, extra)
        wsems = (ws0, ws1)
        iota = lax.iota(jnp.int32, _LANES)
        colq = [16 * q + iota for q in range(8)]
        base4q = [4 * q + iota // 4 for q in range(8)]
        mbase = 32 * jnp.bitwise_and(iota, 3)

        def read(bi):
            s = lax.rem(bi, _RNB)
            pltpu.async_copy(
                tabt_hbm.at[:, pl.ds((start + bi) * 128, 128)],
                vin.at[pl.ds(s * d, d)], rsem.at[s])

        for i in range(_RNB - 1):
            @pl.when(i < nblk)
            def _():
                read(i)

        def wait_write(sem):
            pltpu.make_async_copy(
                vout.at[pl.ds(0, d), pl.ds(0, 128)],
                rt_hbm.at[pl.ds(0, d)], sem).wait()

        def body(bi, carry):
            @pl.when(bi < nblk)
            def _():
                s4 = lax.rem(bi, _RNB)
                s2 = lax.rem(bi, 2)
                pltpu.make_async_copy(
                    tabt_hbm.at[:, pl.ds(0, 128)],
                    vin.at[pl.ds(0, d)], rsem.at[s4]).wait()

                @pl.when(bi + _RNB - 1 < nblk)
                def _():
                    read(bi + _RNB - 1)

                @pl.when((bi >= 2) & (s2 == 0))
                def _():
                    wait_write(ws0)

                @pl.when((bi >= 2) & (s2 == 1))
                def _():
                    wait_write(ws1)

                srowq = [s2 * d + bq for bq in base4q]

                def frow(f, c2):
                    rot = jnp.bitwise_and(f + iota, d - 1)
                    grow = s4 * d + rot
                    scol = mbase + rot
                    for q in range(8):
                        vals = plsc.load_gather(vin, [grow, colq[q]])
                        plsc.store_scatter(vout, [srowq[q], scol], vals)
                    return c2
                lax.fori_loop(0, d, frow, 0, unroll=4)

                @pl.when(s2 == 0)
                def _():
                    pltpu.async_copy(
                        vout.at[pl.ds(0, d), pl.ds(0, 128)],
                        rt_hbm.at[pl.ds((start + bi) * d, d)], ws0)

                @pl.when(s2 == 1)
                def _():
                    pltpu.async_copy(
                        vout.at[pl.ds(d, d), pl.ds(0, 128)],
                        rt_hbm.at[pl.ds((start + bi) * d, d)], ws1)
            return carry

        lax.fori_loop(0, per_w + 1, body, 0)

        wait_write(ws0)
        wait_write(ws1)

        if tail:
            nk = tail * d // 128

            @pl.when(wid == _NW - 1)
            def _():
                pltpu.sync_copy(tail_hbm, vin.at[pl.ds(0, nk)])
                pltpu.sync_copy(vin.at[pl.ds(0, nk)],
                                rt_hbm.at[pl.ds(n_full * d, nk)])

    return tr_kernel


_NBUF = 6


@functools.lru_cache(maxsize=None)
def _make_gather(b, l, v, d):
    assert d == 32 and b % 128 == 0
    nbblk = b // 128
    units = l * nbblk
    assert units % _NW == 0
    per_w = units // _NW
    assert per_w > _NBUF

    @functools.partial(
        pl.kernel,
        mesh=_mesh(),
        out_type=jax.ShapeDtypeStruct((l, d // 8, nbblk, 8, 128),
                                      jnp.float32),
        scratch_types=[
            pltpu.VMEM((per_w * 128,), jnp.int32),
            pltpu.VMEM((_NBUF * 128, d), jnp.float32),
            pltpu.VMEM((2 * d, 130), jnp.float32),
            pltpu.SemaphoreType.DMA((_NBUF,)),
            pltpu.SemaphoreType.DMA,
            pltpu.SemaphoreType.DMA,
        ],
        compiler_params=pltpu.CompilerParams(use_tc_tiling_on_sc=False,
                                             needs_layout_passes=False),
    )
    def g_kernel(tokf_hbm, rt_hbm, out_hbm, ids_v, rows_v, vout,
                 gsem, ws0, ws1):
        wid = _wid()
        u0 = wid * per_w
        wsems = (ws0, ws1)
        iota = lax.iota(jnp.int32, _LANES)
        colq = [16 * q + iota for q in range(8)]

        pltpu.sync_copy(tokf_hbm.at[pl.ds(u0 * 128, per_w * 128)], ids_v)

        def issue(ui):
            s = lax.rem(ui, _NBUF)
            pltpu.async_copy(rt_hbm.at[ids_v.at[pl.ds(ui * 128, 128)]],
                             rows_v.at[pl.ds(s * 128, 128)], gsem.at[s])

        for i in range(_NBUF - 1):
            issue(i)

        def body(ui, carry):
            s6 = lax.rem(ui, _NBUF)
            s2 = lax.rem(ui, 2)
            uu = u0 + ui
            li = uu // nbblk
            tc = lax.rem(uu, nbblk)

            pltpu.make_async_copy(
                rt_hbm.at[ids_v.at[pl.ds(0, 128)]],
                rows_v.at[pl.ds(0, 128)], gsem.at[s6]).wait()

            @pl.when(ui + _NBUF - 1 < per_w)
            def _():
                issue(ui + _NBUF - 1)

            def wait_writes(sem):
                for t in range(d // 8):
                    pltpu.make_async_copy(
                        vout.at[pl.ds(0, 8), pl.ds(0, 128)],
                        out_hbm.at[0, t, 0], sem).wait()

            @pl.when((ui >= 2) & (s2 == 0))
            def _():
                wait_writes(ws0)

            @pl.when((ui >= 2) & (s2 == 1))
            def _():
                wait_writes(ws1)

            rowsq = [s6 * 128 + cq for cq in colq]

            def crow(c, carry2):
                colrot = jnp.bitwise_and(c + iota, d - 1)
                srow = colrot + s2 * d
                for q in range(8):
                    vals = plsc.load_gather(rows_v, [rowsq[q], colrot])
                    plsc.store_scatter(vout, [srow, colq[q]], vals)
                return carry2
            lax.fori_loop(0, d, crow, 0, unroll=4)

            def write(sem):
                for t in range(d // 8):
                    pltpu.async_copy(
                        vout.at[pl.ds(s2 * d + 8 * t, 8), pl.ds(0, 128)],
                        out_hbm.at[li, t, tc], sem)

            @pl.when(s2 == 0)
            def _():
                write(ws0)

            @pl.when(s2 == 1)
            def _():
                write(ws1)
            return carry

        lax.fori_loop(0, per_w, body, 0)

        for s in range(2):
            for t in range(d // 8):
                pltpu.make_async_copy(
                    vout.at[pl.ds(0, 8), pl.ds(0, 128)],
                    out_hbm.at[0, t, 0], wsems[s]).wait()

    return g_kernel


def kernel(token_id, table):
    b, l = token_id.shape
    v, d = table.shape
    if token_id.dtype != jnp.int32:
        token_id = token_id.astype(jnp.int32)
    n_full = v // 128
    tail = v - n_full * 128
    args = [table.T]
    if tail:
        args.append(table[n_full * 128:].reshape(tail * d // 128, 128))
    rt = _make_table_transpose(v, d)(*args)
    rt_lin = rt.reshape(v, d)
    out5 = _make_gather(b, l, v, d)(token_id.T.reshape(b * l), rt_lin)
    return jnp.transpose(out5, (2, 4, 0, 1, 3)).reshape(b, l, d)
